# baseline (speedup 1.0000x reference)
import functools

import jax
import jax.numpy as jnp
from jax import lax
from jax.experimental import pallas as pl
from jax.experimental.pallas import tpu as pltpu
from jax.experimental.pallas import tpu_sc as plsc

_EPS = 1e-15
_L = 16
_NC = 2
_NS = 16
_EB = 128
_Q = 4
_BN = 256
_BE = 2048


def _ceil_to(x, m):
    return (x + m - 1) // m * m


def _gauss_body(pseudo_ref, mu_ref, sg_ref, g_ref, v_ref, *, n_edges, k):
    i = pl.program_id(0)
    p0 = pseudo_ref[:, 0:1]
    p1 = pseudo_ref[:, 1:2]
    mu0 = mu_ref[0:1, :]
    mu1 = mu_ref[1:2, :]
    s0 = sg_ref[0:1, :]
    s1 = sg_ref[1:2, :]
    i0 = 1.0 / (s0 * s0 + _EPS)
    i1 = 1.0 / (s1 * s1 + _EPS)
    d0 = p0 - mu0
    d1 = p1 - mu1
    g = jnp.exp(-0.5 * (d0 * d0 * i0 + d1 * d1 * i1))
    rows = i * _BE + lax.broadcasted_iota(jnp.int32, (_BE, _L), 0)
    cols = lax.broadcasted_iota(jnp.int32, (_BE, _L), 1)
    ev = rows < n_edges
    g_ref[...] = jnp.where(ev & (cols < k), g, 0.0)
    v_ref[...] = jnp.where(ev & (cols == 0), 1.0, 0.0)


def _prep_body(x_ref, w0_ref, w1_ref, w2_ref, w3_ref, wr_ref, b_ref,
               y0_ref, y1_ref, y2_ref, y3_ref, r_ref):
    x = x_ref[...]
    y0_ref[...] = jnp.dot(x, w0_ref[...], preferred_element_type=jnp.float32)
    y1_ref[...] = jnp.dot(x, w1_ref[...], preferred_element_type=jnp.float32)
    y2_ref[...] = jnp.dot(x, w2_ref[...], preferred_element_type=jnp.float32)
    y3_ref[...] = jnp.dot(x, w3_ref[...], preferred_element_type=jnp.float32)
    r_ref[...] = (jnp.dot(x, wr_ref[...], preferred_element_type=jnp.float32)
                  + b_ref[...])


def _combine_body(a0_ref, a1_ref, a2_ref, a3_ref, cnt_ref, root_ref, o_ref):
    agg = jnp.concatenate(
        [a_ref[0] + a_ref[1] for a_ref in (a0_ref, a1_ref, a2_ref, a3_ref)],
        axis=1)
    c = cnt_ref[0][:, 0:1] + cnt_ref[1][:, 0:1]
    h = agg * (1.0 / jnp.maximum(c, 1.0)) + root_ref[...]
    o_ref[...] = jnp.maximum(h, 0.0)


def _combine_res_body(a0_ref, a1_ref, a2_ref, a3_ref, cnt_ref, root_ref,
                      res_ref, o_ref):
    agg = jnp.concatenate(
        [a_ref[0] + a_ref[1] for a_ref in (a0_ref, a1_ref, a2_ref, a3_ref)],
        axis=1)
    c = cnt_ref[0][:, 0:1] + cnt_ref[1][:, 0:1]
    h = agg * (1.0 / jnp.maximum(c, 1.0)) + root_ref[...] + res_ref[...]
    o_ref[...] = jnp.maximum(h, 0.0)


def _sc_quarter_body(yq_ref, src_ref, dst_ref, gauss_ref, zeros_ref, out_ref,
                     sidx, didx, gbuf, rows, stage, acc, sem,
                     *, per_tile, chunks, rpt, k, qw):
    cid = lax.axis_index("c")
    sid = lax.axis_index("s")
    ebase = (cid * _NS + sid) * per_tile
    r0 = sid * rpt
    pltpu.sync_copy(zeros_ref.at[pl.ds(r0, rpt)], acc.at[pl.ds(r0, rpt)])
    plsc.subcore_barrier()

    def chunk(g, carry):
        base = ebase + g * _EB
        pltpu.sync_copy(src_ref.at[pl.ds(base, _EB)], sidx)
        pltpu.sync_copy(dst_ref.at[pl.ds(base, _EB)], didx)
        pltpu.sync_copy(gauss_ref.at[pl.ds(base, _EB)], gbuf)
        pltpu.async_copy(yq_ref.at[sidx], rows, sem).wait()

        def edge(e, c2):
            gv = gbuf[e, pl.ds(0, _L)]
            a0 = gv[0] * rows[e, pl.ds(0, _L)]
            a1 = gv[0] * rows[e, pl.ds(_L, _L)]
            for kk in range(1, k):
                gk = gv[kk]
                a0 = a0 + gk * rows[e, pl.ds(kk * qw, _L)]
                a1 = a1 + gk * rows[e, pl.ds(kk * qw + _L, _L)]
            stage[e, pl.ds(0, _L)] = a0
            stage[e, pl.ds(_L, _L)] = a1
            return c2

        lax.fori_loop(0, _EB, edge, 0)
        pltpu.sync_copy(stage, acc.at[didx], add=True)
        return carry

    lax.fori_loop(0, chunks, chunk, 0)
    plsc.subcore_barrier()
    pltpu.sync_copy(acc.at[pl.ds(r0, rpt)], out_ref.at[cid, pl.ds(r0, rpt)])


def _sc_cnt_body(dst_ref, v16_ref, zeros_ref, out_ref,
                 didx, vbuf, acc, *, per_tile, chunks, rpt):
    cid = lax.axis_index("c")
    sid = lax.axis_index("s")
    ebase = (cid * _NS + sid) * per_tile
    r0 = sid * rpt
    pltpu.sync_copy(zeros_ref.at[pl.ds(r0, rpt)], acc.at[pl.ds(r0, rpt)])
    plsc.subcore_barrier()

    def chunk(g, carry):
        base = ebase + g * _EB
        pltpu.sync_copy(dst_ref.at[pl.ds(base, _EB)], didx)
        pltpu.sync_copy(v16_ref.at[pl.ds(base, _EB)], vbuf)
        pltpu.sync_copy(vbuf, acc.at[didx], add=True)
        return carry

    lax.fori_loop(0, chunks, chunk, 0)
    plsc.subcore_barrier()
    pltpu.sync_copy(acc.at[pl.ds(r0, rpt)], out_ref.at[cid, pl.ds(r0, rpt)])


def kernel(input_feat, edge_index, pseudo, Wg1, mu1, sigma1, Wroot1, b1,
           Wg2, mu2, sigma2, Wroot2, b2):
    n, d = input_feat.shape
    e = pseudo.shape[0]
    k = mu1.shape[0]
    qw = d // _Q
    yw = k * qw

    n_pad = _ceil_to(n, _BN)
    e_pad = _ceil_to(e, _NC * _NS * _EB)
    e_pad = _ceil_to(e_pad, _BE)
    per_tile = e_pad // (_NC * _NS)
    chunks = per_tile // _EB
    rpt = n_pad // _NS

    f32 = jnp.float32

    x_pad = jnp.pad(input_feat, ((0, n_pad - n), (0, 0)))
    src = jnp.pad(edge_index[0], (0, e_pad - e))
    dst = jnp.pad(edge_index[1], (0, e_pad - e))
    pseudo_pad = jnp.pad(pseudo, ((0, e_pad - e), (0, 0)))
    mup1 = jnp.pad(mu1.T, ((0, 0), (0, _L - k)))
    sgp1 = jnp.pad(sigma1.T, ((0, 0), (0, _L - k)), constant_values=1.0)
    mup2 = jnp.pad(mu2.T, ((0, 0), (0, _L - k)))
    sgp2 = jnp.pad(sigma2.T, ((0, 0), (0, _L - k)), constant_values=1.0)
    wq1 = [Wg1.reshape(d, k, _Q, qw)[:, :, q, :].reshape(d, yw)
           for q in range(_Q)]
    wq2 = [Wg2.reshape(d, k, _Q, qw)[:, :, q, :].reshape(d, yw)
           for q in range(_Q)]
    b1r = b1.reshape(1, d)
    b2r = b2.reshape(1, d)
    zeros_q = jnp.zeros((n_pad, qw), f32)
    zeros_c = jnp.zeros((n_pad, _L), f32)

    gauss_call = pl.pallas_call(
        functools.partial(_gauss_body, n_edges=e, k=k),
        grid=(e_pad // _BE,),
        in_specs=[pl.BlockSpec((_BE, 2), lambda i: (i, 0)),
                  pl.BlockSpec((2, _L), lambda i: (0, 0)),
                  pl.BlockSpec((2, _L), lambda i: (0, 0))],
        out_specs=[pl.BlockSpec((_BE, _L), lambda i: (i, 0))] * 2,
        out_shape=[jax.ShapeDtypeStruct((e_pad, _L), f32)] * 2,
    )
    g16_1, v16 = gauss_call(pseudo_pad, mup1, sgp1)
    g16_2, _ = gauss_call(pseudo_pad, mup2, sgp2)

    prep_call = pl.pallas_call(
        _prep_body,
        grid=(n_pad // _BN,),
        in_specs=[pl.BlockSpec((_BN, d), lambda i: (i, 0))] +
                 [pl.BlockSpec((d, yw), lambda i: (0, 0))] * _Q +
                 [pl.BlockSpec((d, d), lambda i: (0, 0)),
                  pl.BlockSpec((1, d), lambda i: (0, 0))],
        out_specs=[pl.BlockSpec((_BN, yw), lambda i: (i, 0))] * _Q +
                  [pl.BlockSpec((_BN, d), lambda i: (i, 0))],
        out_shape=[jax.ShapeDtypeStruct((n_pad, yw), f32)] * _Q +
                  [jax.ShapeDtypeStruct((n_pad, d), f32)],
    )

    mesh = plsc.VectorSubcoreMesh(core_axis_name="c", subcore_axis_name="s",
                                  num_cores=_NC, num_subcores=_NS)
    sc_params = pltpu.CompilerParams(use_tc_tiling_on_sc=False)
    sc_quarter = pl.kernel(
        functools.partial(_sc_quarter_body, per_tile=per_tile, chunks=chunks,
                          rpt=rpt, k=k, qw=qw),
        out_type=jax.ShapeDtypeStruct((_NC, n_pad, qw), f32),
        mesh=mesh,
        scratch_types=[
            pltpu.VMEM((_EB,), jnp.int32),
            pltpu.VMEM((_EB,), jnp.int32),
            pltpu.VMEM((_EB, _L), f32),
            pltpu.VMEM((_EB, yw), f32),
            pltpu.VMEM((_EB, qw), f32),
            pltpu.VMEM_SHARED((n_pad, qw), f32),
            pltpu.SemaphoreType.DMA,
        ],
        compiler_params=sc_params,
    )
    sc_cnt = pl.kernel(
        functools.partial(_sc_cnt_body, per_tile=per_tile, chunks=chunks,
                          rpt=rpt),
        out_type=jax.ShapeDtypeStruct((_NC, n_pad, _L), f32),
        mesh=mesh,
        scratch_types=[
            pltpu.VMEM((_EB,), jnp.int32),
            pltpu.VMEM((_EB, _L), f32),
            pltpu.VMEM_SHARED((n_pad, _L), f32),
        ],
        compiler_params=sc_params,
    )

    combine_call = pl.pallas_call(
        _combine_body,
        grid=(n_pad // _BN,),
        in_specs=[pl.BlockSpec((_NC, _BN, qw), lambda i: (0, i, 0))] * _Q +
                 [pl.BlockSpec((_NC, _BN, _L), lambda i: (0, i, 0)),
                  pl.BlockSpec((_BN, d), lambda i: (i, 0))],
        out_specs=pl.BlockSpec((_BN, d), lambda i: (i, 0)),
        out_shape=jax.ShapeDtypeStruct((n_pad, d), f32),
    )
    combine_res_call = pl.pallas_call(
        _combine_res_body,
        grid=(n_pad // _BN,),
        in_specs=[pl.BlockSpec((_NC, _BN, qw), lambda i: (0, i, 0))] * _Q +
                 [pl.BlockSpec((_NC, _BN, _L), lambda i: (0, i, 0)),
                  pl.BlockSpec((_BN, d), lambda i: (i, 0)),
                  pl.BlockSpec((_BN, d), lambda i: (i, 0))],
        out_specs=pl.BlockSpec((_BN, d), lambda i: (i, 0)),
        out_shape=jax.ShapeDtypeStruct((n_pad, d), f32),
    )

    cnt = sc_cnt(dst, v16, zeros_c)

    y1 = prep_call(x_pad, *wq1, Wroot1, b1r)
    aggs1 = [sc_quarter(y1[q], src, dst, g16_1, zeros_q) for q in range(_Q)]
    h = combine_call(*aggs1, cnt, y1[_Q])

    y2 = prep_call(h, *wq2, Wroot2, b2r)
    aggs2 = [sc_quarter(y2[q], src, dst, g16_2, zeros_q) for q in range(_Q)]
    out = combine_res_call(*aggs2, cnt, y2[_Q], x_pad)

    return out[:n]


# speedup vs baseline: 1.1926x; 1.1926x over previous
import functools

import jax
import jax.numpy as jnp
from jax import lax
from jax.experimental import pallas as pl
from jax.experimental.pallas import tpu as pltpu
from jax.experimental.pallas import tpu_sc as plsc

_EPS = 1e-15
_L = 16
_NC = 2
_NS = 16
_EB = 64
_Q = 4
_BN = 256
_BE = 2048


def _ceil_to(x, m):
    return (x + m - 1) // m * m


def _gauss_body(pseudo_ref, mu_ref, sg_ref, g_ref, v_ref, *, n_edges, k):
    i = pl.program_id(0)
    p0 = pseudo_ref[:, 0:1]
    p1 = pseudo_ref[:, 1:2]
    mu0 = mu_ref[0:1, :]
    mu1 = mu_ref[1:2, :]
    s0 = sg_ref[0:1, :]
    s1 = sg_ref[1:2, :]
    i0 = 1.0 / (s0 * s0 + _EPS)
    i1 = 1.0 / (s1 * s1 + _EPS)
    d0 = p0 - mu0
    d1 = p1 - mu1
    g = jnp.exp(-0.5 * (d0 * d0 * i0 + d1 * d1 * i1))
    rows = i * _BE + lax.broadcasted_iota(jnp.int32, (_BE, _L), 0)
    cols = lax.broadcasted_iota(jnp.int32, (_BE, _L), 1)
    ev = rows < n_edges
    g_ref[...] = jnp.where(ev & (cols < k), g, 0.0)
    v_ref[...] = jnp.where(ev & (cols == 0), 1.0, 0.0)


def _prep_body(x_ref, w0_ref, w1_ref, w2_ref, w3_ref, wr_ref, b_ref,
               y0_ref, y1_ref, y2_ref, y3_ref, r_ref):
    x = x_ref[...]
    y0_ref[...] = jnp.dot(x, w0_ref[...], preferred_element_type=jnp.float32)
    y1_ref[...] = jnp.dot(x, w1_ref[...], preferred_element_type=jnp.float32)
    y2_ref[...] = jnp.dot(x, w2_ref[...], preferred_element_type=jnp.float32)
    y3_ref[...] = jnp.dot(x, w3_ref[...], preferred_element_type=jnp.float32)
    r_ref[...] = (jnp.dot(x, wr_ref[...], preferred_element_type=jnp.float32)
                  + b_ref[...])


def _combine_body(a_ref, cnt_ref, root_ref, o_ref):
    agg = jnp.concatenate([a_ref[q, 0] + a_ref[q, 1] for q in range(_Q)],
                          axis=1)
    c = cnt_ref[0][:, 0:1] + cnt_ref[1][:, 0:1]
    h = agg * (1.0 / jnp.maximum(c, 1.0)) + root_ref[...]
    o_ref[...] = jnp.maximum(h, 0.0)


def _combine_res_body(a_ref, cnt_ref, root_ref, res_ref, o_ref):
    agg = jnp.concatenate([a_ref[q, 0] + a_ref[q, 1] for q in range(_Q)],
                          axis=1)
    c = cnt_ref[0][:, 0:1] + cnt_ref[1][:, 0:1]
    h = agg * (1.0 / jnp.maximum(c, 1.0)) + root_ref[...] + res_ref[...]
    o_ref[...] = jnp.maximum(h, 0.0)


def _sc_layer_body(y0_ref, y1_ref, y2_ref, y3_ref, src_ref, dst_ref,
                   gauss_ref, zeros_ref, out_ref,
                   sidx0, sidx1, didx0, didx1, gbuf0, gbuf1,
                   rows0, rows1, stage0, stage1, acc, sem0, sem1,
                   *, per_tile, chunks, rpt, k, qw):
    cid = lax.axis_index("c")
    sid = lax.axis_index("s")
    ebase = (cid * _NS + sid) * per_tile
    r0 = sid * rpt
    sidx = (sidx0, sidx1)
    didx = (didx0, didx1)
    gbuf = (gbuf0, gbuf1)
    rows = (rows0, rows1)
    stage = (stage0, stage1)
    sem = (sem0, sem1)
    last = chunks - 1

    for q, yq_ref in enumerate((y0_ref, y1_ref, y2_ref, y3_ref)):
        pltpu.sync_copy(zeros_ref.at[pl.ds(r0, rpt)], acc.at[pl.ds(r0, rpt)])
        plsc.subcore_barrier()

        def fetch(g, p):
            base = ebase + g * _EB
            pltpu.sync_copy(src_ref.at[pl.ds(base, _EB)], sidx[p])
            pltpu.sync_copy(dst_ref.at[pl.ds(base, _EB)], didx[p])
            pltpu.sync_copy(gauss_ref.at[pl.ds(base, _EB)], gbuf[p])
            return pltpu.async_copy(yq_ref.at[sidx[p]], rows[p], sem[p])

        def compute(p):
            rw = rows[p]
            st = stage[p]
            gb = gbuf[p]

            def edge(e, c2):
                gv = gb[e, pl.ds(0, _L)]
                a0 = gv[0] * rw[e, pl.ds(0, _L)]
                a1 = gv[0] * rw[e, pl.ds(_L, _L)]
                for kk in range(1, k):
                    gk = gv[kk]
                    a0 = a0 + gk * rw[e, pl.ds(kk * qw, _L)]
                    a1 = a1 + gk * rw[e, pl.ds(kk * qw + _L, _L)]
                st[e, pl.ds(0, _L)] = a0
                st[e, pl.ds(_L, _L)] = a1
                return c2

            lax.fori_loop(0, _EB, edge, 0)
            pltpu.sync_copy(st, acc.at[didx[p]], add=True)

        fetch(0, 0)

        def wait_gather(p):
            pltpu.make_async_copy(yq_ref.at[sidx[p]], rows[p], sem[p]).wait()

        def pair(j, carry):
            fetch(2 * j + 1, 1)
            wait_gather(0)
            compute(0)
            fetch(jnp.minimum(2 * j + 2, last), 0)
            wait_gather(1)
            compute(1)
            return carry

        lax.fori_loop(0, chunks // 2, pair, 0)
        wait_gather(0)
        plsc.subcore_barrier()
        pltpu.sync_copy(acc.at[pl.ds(r0, rpt)],
                        out_ref.at[q, cid, pl.ds(r0, rpt)])
        plsc.subcore_barrier()


def _sc_cnt_body(dst_ref, v16_ref, zeros_ref, out_ref,
                 didx, vbuf, acc, *, per_tile, chunks, rpt):
    cid = lax.axis_index("c")
    sid = lax.axis_index("s")
    ebase = (cid * _NS + sid) * per_tile
    r0 = sid * rpt
    pltpu.sync_copy(zeros_ref.at[pl.ds(r0, rpt)], acc.at[pl.ds(r0, rpt)])
    plsc.subcore_barrier()

    def chunk(g, carry):
        base = ebase + g * _EB
        pltpu.sync_copy(dst_ref.at[pl.ds(base, _EB)], didx)
        pltpu.sync_copy(v16_ref.at[pl.ds(base, _EB)], vbuf)
        pltpu.sync_copy(vbuf, acc.at[didx], add=True)
        return carry

    lax.fori_loop(0, chunks, chunk, 0)
    plsc.subcore_barrier()
    pltpu.sync_copy(acc.at[pl.ds(r0, rpt)], out_ref.at[cid, pl.ds(r0, rpt)])


def kernel(input_feat, edge_index, pseudo, Wg1, mu1, sigma1, Wroot1, b1,
           Wg2, mu2, sigma2, Wroot2, b2):
    n, d = input_feat.shape
    e = pseudo.shape[0]
    k = mu1.shape[0]
    qw = d // _Q
    yw = k * qw

    n_pad = _ceil_to(n, _BN)
    e_pad = _ceil_to(e, _NC * _NS * _EB * 2)
    e_pad = _ceil_to(e_pad, _BE)
    per_tile = e_pad // (_NC * _NS)
    chunks = per_tile // _EB
    rpt = n_pad // _NS

    f32 = jnp.float32

    x_pad = jnp.pad(input_feat, ((0, n_pad - n), (0, 0)))
    src = jnp.pad(edge_index[0], (0, e_pad - e))
    dst = jnp.pad(edge_index[1], (0, e_pad - e))
    pseudo_pad = jnp.pad(pseudo, ((0, e_pad - e), (0, 0)))
    mup1 = jnp.pad(mu1.T, ((0, 0), (0, _L - k)))
    sgp1 = jnp.pad(sigma1.T, ((0, 0), (0, _L - k)), constant_values=1.0)
    mup2 = jnp.pad(mu2.T, ((0, 0), (0, _L - k)))
    sgp2 = jnp.pad(sigma2.T, ((0, 0), (0, _L - k)), constant_values=1.0)
    wq1 = [Wg1.reshape(d, k, _Q, qw)[:, :, q, :].reshape(d, yw)
           for q in range(_Q)]
    wq2 = [Wg2.reshape(d, k, _Q, qw)[:, :, q, :].reshape(d, yw)
           for q in range(_Q)]
    b1r = b1.reshape(1, d)
    b2r = b2.reshape(1, d)
    zeros_q = jnp.zeros((n_pad, qw), f32)
    zeros_c = jnp.zeros((n_pad, _L), f32)

    gauss_call = pl.pallas_call(
        functools.partial(_gauss_body, n_edges=e, k=k),
        grid=(e_pad // _BE,),
        in_specs=[pl.BlockSpec((_BE, 2), lambda i: (i, 0)),
                  pl.BlockSpec((2, _L), lambda i: (0, 0)),
                  pl.BlockSpec((2, _L), lambda i: (0, 0))],
        out_specs=[pl.BlockSpec((_BE, _L), lambda i: (i, 0))] * 2,
        out_shape=[jax.ShapeDtypeStruct((e_pad, _L), f32)] * 2,
    )
    g16_1, v16 = gauss_call(pseudo_pad, mup1, sgp1)
    g16_2, _ = gauss_call(pseudo_pad, mup2, sgp2)

    prep_call = pl.pallas_call(
        _prep_body,
        grid=(n_pad // _BN,),
        in_specs=[pl.BlockSpec((_BN, d), lambda i: (i, 0))] +
                 [pl.BlockSpec((d, yw), lambda i: (0, 0))] * _Q +
                 [pl.BlockSpec((d, d), lambda i: (0, 0)),
                  pl.BlockSpec((1, d), lambda i: (0, 0))],
        out_specs=[pl.BlockSpec((_BN, yw), lambda i: (i, 0))] * _Q +
                  [pl.BlockSpec((_BN, d), lambda i: (i, 0))],
        out_shape=[jax.ShapeDtypeStruct((n_pad, yw), f32)] * _Q +
                  [jax.ShapeDtypeStruct((n_pad, d), f32)],
    )

    mesh = plsc.VectorSubcoreMesh(core_axis_name="c", subcore_axis_name="s",
                                  num_cores=_NC, num_subcores=_NS)
    sc_params = pltpu.CompilerParams(use_tc_tiling_on_sc=False)
    sc_layer = pl.kernel(
        functools.partial(_sc_layer_body, per_tile=per_tile, chunks=chunks,
                          rpt=rpt, k=k, qw=qw),
        out_type=jax.ShapeDtypeStruct((_Q, _NC, n_pad, qw), f32),
        mesh=mesh,
        scratch_types=[
            pltpu.VMEM((_EB,), jnp.int32),
            pltpu.VMEM((_EB,), jnp.int32),
            pltpu.VMEM((_EB,), jnp.int32),
            pltpu.VMEM((_EB,), jnp.int32),
            pltpu.VMEM((_EB, _L), f32),
            pltpu.VMEM((_EB, _L), f32),
            pltpu.VMEM((_EB, yw), f32),
            pltpu.VMEM((_EB, yw), f32),
            pltpu.VMEM((_EB, qw), f32),
            pltpu.VMEM((_EB, qw), f32),
            pltpu.VMEM_SHARED((n_pad, qw), f32),
            pltpu.SemaphoreType.DMA,
            pltpu.SemaphoreType.DMA,
        ],
        compiler_params=sc_params,
    )
    sc_cnt = pl.kernel(
        functools.partial(_sc_cnt_body, per_tile=per_tile, chunks=chunks,
                          rpt=rpt),
        out_type=jax.ShapeDtypeStruct((_NC, n_pad, _L), f32),
        mesh=mesh,
        scratch_types=[
            pltpu.VMEM((_EB,), jnp.int32),
            pltpu.VMEM((_EB, _L), f32),
            pltpu.VMEM_SHARED((n_pad, _L), f32),
        ],
        compiler_params=sc_params,
    )

    combine_call = pl.pallas_call(
        _combine_body,
        grid=(n_pad // _BN,),
        in_specs=[pl.BlockSpec((_Q, _NC, _BN, qw), lambda i: (0, 0, i, 0)),
                  pl.BlockSpec((_NC, _BN, _L), lambda i: (0, i, 0)),
                  pl.BlockSpec((_BN, d), lambda i: (i, 0))],
        out_specs=pl.BlockSpec((_BN, d), lambda i: (i, 0)),
        out_shape=jax.ShapeDtypeStruct((n_pad, d), f32),
    )
    combine_res_call = pl.pallas_call(
        _combine_res_body,
        grid=(n_pad // _BN,),
        in_specs=[pl.BlockSpec((_Q, _NC, _BN, qw), lambda i: (0, 0, i, 0)),
                  pl.BlockSpec((_NC, _BN, _L), lambda i: (0, i, 0)),
                  pl.BlockSpec((_BN, d), lambda i: (i, 0)),
                  pl.BlockSpec((_BN, d), lambda i: (i, 0))],
        out_specs=pl.BlockSpec((_BN, d), lambda i: (i, 0)),
        out_shape=jax.ShapeDtypeStruct((n_pad, d), f32),
    )

    cnt = sc_cnt(dst, v16, zeros_c)

    y1 = prep_call(x_pad, *wq1, Wroot1, b1r)
    aggs1 = sc_layer(y1[0], y1[1], y1[2], y1[3], src, dst, g16_1, zeros_q)
    h = combine_call(aggs1, cnt, y1[_Q])

    y2 = prep_call(h, *wq2, Wroot2, b2r)
    aggs2 = sc_layer(y2[0], y2[1], y2[2], y2[3], src, dst, g16_2, zeros_q)
    out = combine_res_call(aggs2, cnt, y2[_Q], x_pad)

    return out[:n]


# speedup vs baseline: 1.3044x; 1.0937x over previous
import functools

import jax
import jax.numpy as jnp
from jax import lax
from jax.experimental import pallas as pl
from jax.experimental.pallas import tpu as pltpu
from jax.experimental.pallas import tpu_sc as plsc

_EPS = 1e-15
_L = 16
_NC = 2
_NS = 16
_EB = 64
_Q = 4
_BN = 256
_BE = 2048


def _ceil_to(x, m):
    return (x + m - 1) // m * m


def _gauss_body(pseudo_ref, mu_ref, sg_ref, g_ref, v_ref, *, n_edges, k):
    i = pl.program_id(0)
    p0 = pseudo_ref[:, 0:1]
    p1 = pseudo_ref[:, 1:2]
    mu0 = mu_ref[0:1, :]
    mu1 = mu_ref[1:2, :]
    s0 = sg_ref[0:1, :]
    s1 = sg_ref[1:2, :]
    i0 = 1.0 / (s0 * s0 + _EPS)
    i1 = 1.0 / (s1 * s1 + _EPS)
    d0 = p0 - mu0
    d1 = p1 - mu1
    g = jnp.exp(-0.5 * (d0 * d0 * i0 + d1 * d1 * i1))
    rows = i * _BE + lax.broadcasted_iota(jnp.int32, (_BE, _L), 0)
    cols = lax.broadcasted_iota(jnp.int32, (_BE, _L), 1)
    ev = rows < n_edges
    g_ref[...] = jnp.where(ev & (cols < k), g, 0.0)
    v_ref[...] = jnp.where(ev & (cols == 0), 1.0, 0.0)


def _prep_body(x_ref, w0_ref, w1_ref, w2_ref, w3_ref, wr_ref, b_ref,
               y0_ref, y1_ref, y2_ref, y3_ref, r_ref):
    x = x_ref[...]
    for w_ref, y_ref in ((w0_ref, y0_ref), (w1_ref, y1_ref),
                         (w2_ref, y2_ref), (w3_ref, y3_ref)):
        y = jnp.dot(x, w_ref[...], preferred_element_type=jnp.float32)
        y_ref[...] = y.astype(jnp.bfloat16)
    r_ref[...] = (jnp.dot(x, wr_ref[...], preferred_element_type=jnp.float32)
                  + b_ref[...])


def _combine_body(a_ref, cnt_ref, root_ref, o_ref):
    agg = jnp.concatenate([a_ref[q, 0] + a_ref[q, 1] for q in range(_Q)],
                          axis=1)
    c = cnt_ref[0][:, 0:1] + cnt_ref[1][:, 0:1]
    h = agg * (1.0 / jnp.maximum(c, 1.0)) + root_ref[...]
    o_ref[...] = jnp.maximum(h, 0.0)


def _combine_res_body(a_ref, cnt_ref, root_ref, res_ref, o_ref):
    agg = jnp.concatenate([a_ref[q, 0] + a_ref[q, 1] for q in range(_Q)],
                          axis=1)
    c = cnt_ref[0][:, 0:1] + cnt_ref[1][:, 0:1]
    h = agg * (1.0 / jnp.maximum(c, 1.0)) + root_ref[...] + res_ref[...]
    o_ref[...] = jnp.maximum(h, 0.0)


def _sc_layer_body(y0_ref, y1_ref, y2_ref, y3_ref, src2_ref, dst2_ref,
                   gauss_ref, zeros_ref, out_ref,
                   sidx_all, didx_all, gbuf0, gbuf1,
                   rows0, rows1, stage0, stage1, acc,
                   gsem0, gsem1, sem0, sem1,
                   *, per_tile, chunks, rpt, k, qw):
    cid = lax.axis_index("c")
    sid = lax.axis_index("s")
    tid = cid * _NS + sid
    ebase = tid * per_tile
    r0 = sid * rpt
    gbuf = (gbuf0, gbuf1)
    rows = (rows0, rows1)
    stage = (stage0, stage1)
    gsem = (gsem0, gsem1)
    sem = (sem0, sem1)
    last = chunks - 1

    pltpu.sync_copy(src2_ref.at[pl.ds(tid * chunks, chunks)], sidx_all)
    pltpu.sync_copy(dst2_ref.at[pl.ds(tid * chunks, chunks)], didx_all)

    for q, yq_ref in enumerate((y0_ref, y1_ref, y2_ref, y3_ref)):
        pltpu.sync_copy(zeros_ref.at[pl.ds(r0, rpt)], acc.at[pl.ds(r0, rpt)])
        plsc.subcore_barrier()

        def prefetch(g, p):
            pltpu.async_copy(gauss_ref.at[pl.ds(ebase + g * _EB, _EB)],
                             gbuf[p], gsem[p])
            pltpu.async_copy(yq_ref.at[sidx_all.at[g]], rows[p], sem[p])

        def wait_prefetch(p):
            pltpu.make_async_copy(gauss_ref.at[pl.ds(ebase, _EB)],
                                  gbuf[p], gsem[p]).wait()
            pltpu.make_async_copy(yq_ref.at[sidx_all.at[0]],
                                  rows[p], sem[p]).wait()

        def compute(g, p):
            rw = rows[p]
            st = stage[p]
            gb = gbuf[p]

            def edge(e, c2):
                gv = gb[e, pl.ds(0, _L)]
                a0 = jnp.zeros((_L,), jnp.float32)
                a1 = jnp.zeros((_L,), jnp.float32)
                for kk in range(k):
                    w = plsc.bitcast(rw[e, pl.ds(kk * qw, qw)], jnp.int32)
                    fe = plsc.bitcast(w << 16, jnp.float32)
                    fo = plsc.bitcast(w & jnp.int32(-65536), jnp.float32)
                    gk = gv[kk]
                    a0 = a0 + gk * fe
                    a1 = a1 + gk * fo
                st[e, pl.ds(0, _L)] = a0
                st[e, pl.ds(_L, _L)] = a1
                return c2

            lax.fori_loop(0, _EB, edge, 0)
            pltpu.sync_copy(st, acc.at[didx_all.at[g]], add=True)

        prefetch(0, 0)
        prefetch(1, 1)

        def pair(j, carry):
            g0 = 2 * j
            wait_prefetch(0)
            compute(g0, 0)
            prefetch(jnp.minimum(g0 + 2, last), 0)
            wait_prefetch(1)
            compute(g0 + 1, 1)
            prefetch(jnp.minimum(g0 + 3, last), 1)
            return carry

        lax.fori_loop(0, chunks // 2, pair, 0)
        wait_prefetch(0)
        wait_prefetch(1)
        plsc.subcore_barrier()
        pltpu.sync_copy(acc.at[pl.ds(r0, rpt)],
                        out_ref.at[q, cid, pl.ds(r0, rpt)])
        plsc.subcore_barrier()


def _sc_cnt_body(dst2_ref, v16_ref, zeros_ref, out_ref,
                 didx_all, vbuf, acc, *, per_tile, chunks, rpt):
    cid = lax.axis_index("c")
    sid = lax.axis_index("s")
    tid = cid * _NS + sid
    ebase = tid * per_tile
    r0 = sid * rpt
    pltpu.sync_copy(zeros_ref.at[pl.ds(r0, rpt)], acc.at[pl.ds(r0, rpt)])
    pltpu.sync_copy(dst2_ref.at[pl.ds(tid * chunks, chunks)], didx_all)
    plsc.subcore_barrier()

    def chunk(g, carry):
        base = ebase + g * _EB
        pltpu.sync_copy(v16_ref.at[pl.ds(base, _EB)], vbuf)
        pltpu.sync_copy(vbuf, acc.at[didx_all.at[g]], add=True)
        return carry

    lax.fori_loop(0, chunks, chunk, 0)
    plsc.subcore_barrier()
    pltpu.sync_copy(acc.at[pl.ds(r0, rpt)], out_ref.at[cid, pl.ds(r0, rpt)])


def kernel(input_feat, edge_index, pseudo, Wg1, mu1, sigma1, Wroot1, b1,
           Wg2, mu2, sigma2, Wroot2, b2):
    n, d = input_feat.shape
    e = pseudo.shape[0]
    k = mu1.shape[0]
    qw = d // _Q
    yw = k * qw

    n_pad = _ceil_to(n, _BN)
    e_pad = _ceil_to(e, _NC * _NS * _EB * 2)
    e_pad = _ceil_to(e_pad, _BE)
    per_tile = e_pad // (_NC * _NS)
    chunks = per_tile // _EB
    rpt = n_pad // _NS

    f32 = jnp.float32

    x_pad = jnp.pad(input_feat, ((0, n_pad - n), (0, 0)))
    src = jnp.pad(edge_index[0], (0, e_pad - e)).reshape(-1, _EB)
    dst = jnp.pad(edge_index[1], (0, e_pad - e)).reshape(-1, _EB)
    pseudo_pad = jnp.pad(pseudo, ((0, e_pad - e), (0, 0)))
    mup1 = jnp.pad(mu1.T, ((0, 0), (0, _L - k)))
    sgp1 = jnp.pad(sigma1.T, ((0, 0), (0, _L - k)), constant_values=1.0)
    mup2 = jnp.pad(mu2.T, ((0, 0), (0, _L - k)))
    sgp2 = jnp.pad(sigma2.T, ((0, 0), (0, _L - k)), constant_values=1.0)
    def _perm(wg):
        blk = wg.reshape(d, k, _Q, qw)
        il = jnp.stack([blk[..., :qw // 2], blk[..., qw // 2:]], axis=-1)
        il = il.reshape(d, k, _Q, qw)
        return [il[:, :, q, :].reshape(d, yw) for q in range(_Q)]

    wq1 = _perm(Wg1)
    wq2 = _perm(Wg2)
    b1r = b1.reshape(1, d)
    b2r = b2.reshape(1, d)
    zeros_q = jnp.zeros((n_pad, qw), f32)
    zeros_c = jnp.zeros((n_pad, _L), f32)

    gauss_call = pl.pallas_call(
        functools.partial(_gauss_body, n_edges=e, k=k),
        grid=(e_pad // _BE,),
        in_specs=[pl.BlockSpec((_BE, 2), lambda i: (i, 0)),
                  pl.BlockSpec((2, _L), lambda i: (0, 0)),
                  pl.BlockSpec((2, _L), lambda i: (0, 0))],
        out_specs=[pl.BlockSpec((_BE, _L), lambda i: (i, 0))] * 2,
        out_shape=[jax.ShapeDtypeStruct((e_pad, _L), f32)] * 2,
    )
    g16_1, v16 = gauss_call(pseudo_pad, mup1, sgp1)
    g16_2, _ = gauss_call(pseudo_pad, mup2, sgp2)

    prep_call = pl.pallas_call(
        _prep_body,
        grid=(n_pad // _BN,),
        in_specs=[pl.BlockSpec((_BN, d), lambda i: (i, 0))] +
                 [pl.BlockSpec((d, yw), lambda i: (0, 0))] * _Q +
                 [pl.BlockSpec((d, d), lambda i: (0, 0)),
                  pl.BlockSpec((1, d), lambda i: (0, 0))],
        out_specs=[pl.BlockSpec((_BN, yw), lambda i: (i, 0))] * _Q +
                  [pl.BlockSpec((_BN, d), lambda i: (i, 0))],
        out_shape=[jax.ShapeDtypeStruct((n_pad, yw), jnp.bfloat16)] * _Q +
                  [jax.ShapeDtypeStruct((n_pad, d), f32)],
    )

    mesh = plsc.VectorSubcoreMesh(core_axis_name="c", subcore_axis_name="s",
                                  num_cores=_NC, num_subcores=_NS)
    sc_params = pltpu.CompilerParams(use_tc_tiling_on_sc=False,
                                     needs_layout_passes=False)
    sc_layer = pl.kernel(
        functools.partial(_sc_layer_body, per_tile=per_tile, chunks=chunks,
                          rpt=rpt, k=k, qw=qw),
        out_type=jax.ShapeDtypeStruct((_Q, _NC, n_pad, qw), f32),
        mesh=mesh,
        scratch_types=[
            pltpu.VMEM((chunks, _EB), jnp.int32),
            pltpu.VMEM((chunks, _EB), jnp.int32),
            pltpu.VMEM((_EB, _L), f32),
            pltpu.VMEM((_EB, _L), f32),
            pltpu.VMEM((_EB, yw), jnp.bfloat16),
            pltpu.VMEM((_EB, yw), jnp.bfloat16),
            pltpu.VMEM((_EB, qw), f32),
            pltpu.VMEM((_EB, qw), f32),
            pltpu.VMEM_SHARED((n_pad, qw), f32),
            pltpu.SemaphoreType.DMA,
            pltpu.SemaphoreType.DMA,
            pltpu.SemaphoreType.DMA,
            pltpu.SemaphoreType.DMA,
        ],
        compiler_params=sc_params,
    )
    sc_cnt = pl.kernel(
        functools.partial(_sc_cnt_body, per_tile=per_tile, chunks=chunks,
                          rpt=rpt),
        out_type=jax.ShapeDtypeStruct((_NC, n_pad, _L), f32),
        mesh=mesh,
        scratch_types=[
            pltpu.VMEM((chunks, _EB), jnp.int32),
            pltpu.VMEM((_EB, _L), f32),
            pltpu.VMEM_SHARED((n_pad, _L), f32),
        ],
        compiler_params=sc_params,
    )

    combine_call = pl.pallas_call(
        _combine_body,
        grid=(n_pad // _BN,),
        in_specs=[pl.BlockSpec((_Q, _NC, _BN, qw), lambda i: (0, 0, i, 0)),
                  pl.BlockSpec((_NC, _BN, _L), lambda i: (0, i, 0)),
                  pl.BlockSpec((_BN, d), lambda i: (i, 0))],
        out_specs=pl.BlockSpec((_BN, d), lambda i: (i, 0)),
        out_shape=jax.ShapeDtypeStruct((n_pad, d), f32),
    )
    combine_res_call = pl.pallas_call(
        _combine_res_body,
        grid=(n_pad // _BN,),
        in_specs=[pl.BlockSpec((_Q, _NC, _BN, qw), lambda i: (0, 0, i, 0)),
                  pl.BlockSpec((_NC, _BN, _L), lambda i: (0, i, 0)),
                  pl.BlockSpec((_BN, d), lambda i: (i, 0)),
                  pl.BlockSpec((_BN, d), lambda i: (i, 0))],
        out_specs=pl.BlockSpec((_BN, d), lambda i: (i, 0)),
        out_shape=jax.ShapeDtypeStruct((n_pad, d), f32),
    )

    cnt = sc_cnt(dst, v16, zeros_c)

    y1 = prep_call(x_pad, *wq1, Wroot1, b1r)
    aggs1 = sc_layer(y1[0], y1[1], y1[2], y1[3], src, dst, g16_1, zeros_q)
    h = combine_call(aggs1, cnt, y1[_Q])

    y2 = prep_call(h, *wq2, Wroot2, b2r)
    aggs2 = sc_layer(y2[0], y2[1], y2[2], y2[3], src, dst, g16_2, zeros_q)
    out = combine_res_call(aggs2, cnt, y2[_Q], x_pad)

    return out[:n]


# speedup vs baseline: 1.5549x; 1.1920x over previous
import functools

import jax
import jax.numpy as jnp
from jax import lax
from jax.experimental import pallas as pl
from jax.experimental.pallas import tpu as pltpu
from jax.experimental.pallas import tpu_sc as plsc

_EPS = 1e-15
_L = 16
_NC = 2
_NS = 16
_EB = 64
_Q = 4
_BN = 256
_BE = 2048


def _ceil_to(x, m):
    return (x + m - 1) // m * m


def _gauss_body(p0_ref, p1_ref, mu_ref, sg_ref, g_ref, *, k):
    p0 = p0_ref[...]
    p1 = p1_ref[...]
    mu0 = mu_ref[0:1, :]
    mu1 = mu_ref[1:2, :]
    s0 = sg_ref[0:1, :]
    s1 = sg_ref[1:2, :]
    i0 = 1.0 / (s0 * s0 + _EPS)
    i1 = 1.0 / (s1 * s1 + _EPS)
    d0 = p0 - mu0
    d1 = p1 - mu1
    g = jnp.exp(-0.5 * (d0 * d0 * i0 + d1 * d1 * i1))
    cols = lax.broadcasted_iota(jnp.int32, g.shape, 1) % _L
    g_ref[...] = jnp.where(cols < k, g, 0.0)


def _prep_body(x_ref, w0_ref, w1_ref, w2_ref, w3_ref, wr_ref, b_ref,
               y0_ref, y1_ref, y2_ref, y3_ref, r_ref):
    x = x_ref[...]
    for w_ref, y_ref in ((w0_ref, y0_ref), (w1_ref, y1_ref),
                         (w2_ref, y2_ref), (w3_ref, y3_ref)):
        y = jnp.dot(x, w_ref[...], preferred_element_type=jnp.float32)
        y_ref[...] = y.astype(jnp.bfloat16)
    r_ref[...] = (jnp.dot(x, wr_ref[...], preferred_element_type=jnp.float32)
                  + b_ref[...])


def _sc_fin_body(aggs_ref, cnt_ref, root_ref, res_ref, h_ref,
                 abuf0, abuf1, cbuf0, cbuf1, rbuf0, rbuf1,
                 sbuf0, sbuf1, hbuf0, hbuf1, sem0, sem1,
                 *, rpt32, qw, with_res):
    cid = lax.axis_index("c")
    sid = lax.axis_index("s")
    r0 = (cid * _NS + sid) * rpt32
    rb = _EB
    nch = rpt32 // rb
    abuf = (abuf0, abuf1)
    cbuf = (cbuf0, cbuf1)
    rbuf = (rbuf0, rbuf1)
    sbuf = (sbuf0, sbuf1)
    hbuf = (hbuf0, hbuf1)
    sem = (sem0, sem1)

    def issue(j, p):
        rbase = r0 + j * rb
        for q in range(_Q):
            for c in range(_NC):
                pltpu.async_copy(aggs_ref.at[q, c, pl.ds(rbase, rb)],
                                 abuf[p].at[q * _NC + c], sem[p])
        for c in range(_NC):
            pltpu.async_copy(cnt_ref.at[c, pl.ds(rbase, rb)],
                             cbuf[p].at[c], sem[p])
        pltpu.async_copy(root_ref.at[pl.ds(rbase, rb)], rbuf[p], sem[p])
        if with_res:
            pltpu.async_copy(res_ref.at[pl.ds(rbase, rb)], sbuf[p], sem[p])

    def wait(p):
        for q2 in range(_Q * _NC):
            pltpu.make_async_copy(aggs_ref.at[0, 0, pl.ds(r0, rb)],
                                  abuf[p].at[q2], sem[p]).wait()
        for c in range(_NC):
            pltpu.make_async_copy(cnt_ref.at[0, pl.ds(r0, rb)],
                                  cbuf[p].at[c], sem[p]).wait()
        pltpu.make_async_copy(root_ref.at[pl.ds(r0, rb)],
                              rbuf[p], sem[p]).wait()
        if with_res:
            pltpu.make_async_copy(res_ref.at[pl.ds(r0, rb)],
                                  sbuf[p], sem[p]).wait()

    def compute(p):
        ab = abuf[p]
        cb = cbuf[p]
        rbf = rbuf[p]
        sb = sbuf[p]
        hb = hbuf[p]

        def row(r, c2):
            cv = cb[0, r, pl.ds(0, _L)] + cb[1, r, pl.ds(0, _L)]
            inv = 1.0 / jnp.maximum(cv, 1.0)
            iv = inv[0]
            for q in range(_Q):
                for hh in range(qw // _L):
                    col = q * qw + hh * _L
                    a = (ab[q * _NC, r, pl.ds(hh * _L, _L)]
                         + ab[q * _NC + 1, r, pl.ds(hh * _L, _L)])
                    v = a * iv + rbf[r, pl.ds(col, _L)]
                    if with_res:
                        v = v + sb[r, pl.ds(col, _L)]
                    hb[r, pl.ds(col, _L)] = jnp.maximum(v, 0.0)
            return c2

        lax.fori_loop(0, rb, row, 0)

    def flush(j, p):
        pltpu.sync_copy(hbuf[p], h_ref.at[pl.ds(r0 + j * rb, rb)])

    issue(0, 0)

    def pair(i, carry):
        j = 2 * i
        issue(j + 1, 1)
        wait(0)
        compute(0)
        flush(j, 0)
        issue(jnp.minimum(j + 2, nch - 1), 0)
        wait(1)
        compute(1)
        flush(j + 1, 1)
        return carry

    lax.fori_loop(0, nch // 2, pair, 0)
    if nch % 2:
        wait(0)
        compute(0)
        flush(nch - 1, 0)
    else:
        wait(0)


def _zero_slice(zbuf, acc, r0, rpt, w16):
    def zrow(i, c):
        for j in range(w16):
            zbuf[i, pl.ds(j * _L, _L)] = jnp.zeros((_L,), jnp.float32)
        return c

    lax.fori_loop(0, _EB, zrow, 0)

    def zcopy(j, c):
        pltpu.sync_copy(zbuf, acc.at[pl.ds(r0 + j * _EB, _EB)])
        return c

    nfull = rpt // _EB
    lax.fori_loop(0, nfull, zcopy, 0)
    rem = rpt - nfull * _EB
    if rem:
        pltpu.sync_copy(zbuf.at[pl.ds(0, rem)],
                        acc.at[pl.ds(r0 + nfull * _EB, rem)])


def _sc_layer_body(y0_ref, y1_ref, y2_ref, y3_ref, src1_ref, dst1_ref,
                   gauss_ref, out_ref,
                   sidx_all, didx_all, gbuf0, gbuf1,
                   rows0, rows1, stage, zbuf, acc,
                   gsem0, gsem1, sem0, sem1,
                   *, per_tile, chunks, rpt, k, qw):
    cid = lax.axis_index("c")
    sid = lax.axis_index("s")
    tid = cid * _NS + sid
    ebase = tid * per_tile
    r0 = sid * rpt
    gbuf = (gbuf0, gbuf1)
    rows = (rows0, rows1)
    gsem = (gsem0, gsem1)
    sem = (sem0, sem1)
    last = chunks - 1

    pt128 = per_tile // (2 * _EB)
    pltpu.sync_copy(src1_ref.at[pl.ds(tid * pt128, pt128)], sidx_all)
    pltpu.sync_copy(dst1_ref.at[pl.ds(tid * pt128, pt128)], didx_all)
    gb8 = _EB // 8

    for q, yq_ref in enumerate((y0_ref, y1_ref, y2_ref, y3_ref)):
        _zero_slice(zbuf, acc, r0, rpt, qw // _L)
        plsc.subcore_barrier()

        def prefetch(g, p):
            pltpu.async_copy(
                gauss_ref.at[pl.ds((ebase + g * _EB) // 8, gb8)],
                gbuf[p], gsem[p])
            pltpu.async_copy(
                yq_ref.at[sidx_all.at[g >> 1, pl.ds((g & 1) * _EB, _EB)]],
                rows[p], sem[p])

        def wait_prefetch(p):
            pltpu.make_async_copy(gauss_ref.at[pl.ds(0, gb8)],
                                  gbuf[p], gsem[p]).wait()
            pltpu.make_async_copy(
                yq_ref.at[sidx_all.at[0, pl.ds(0, _EB)]],
                rows[p], sem[p]).wait()

        def compute(p, half):
            rw = rows[p]
            gb = gbuf[p]
            base = half * _EB

            def edge(e, c2):
                gv = gb[e >> 3, pl.ds((e & 7) * _L, _L)]
                a0 = jnp.zeros((_L,), jnp.float32)
                a1 = jnp.zeros((_L,), jnp.float32)
                for kk in range(k):
                    w = plsc.bitcast(rw[e, pl.ds(kk * qw, qw)], jnp.int32)
                    fe = plsc.bitcast(w << 16, jnp.float32)
                    fo = plsc.bitcast(w & jnp.int32(-65536), jnp.float32)
                    gk = gv[kk]
                    a0 = a0 + gk * fe
                    a1 = a1 + gk * fo
                stage[base + e, pl.ds(0, _L)] = a0
                stage[base + e, pl.ds(_L, _L)] = a1
                return c2

            lax.fori_loop(0, _EB, edge, 0)

        prefetch(0, 0)
        prefetch(1, 1)

        def pair(j, carry):
            g0 = 2 * j
            wait_prefetch(0)
            compute(0, 0)
            prefetch(jnp.minimum(g0 + 2, last), 0)
            wait_prefetch(1)
            compute(1, 1)
            prefetch(jnp.minimum(g0 + 3, last), 1)
            pltpu.sync_copy(stage, acc.at[didx_all.at[j]], add=True)
            return carry

        lax.fori_loop(0, chunks // 2, pair, 0)
        wait_prefetch(0)
        wait_prefetch(1)
        plsc.subcore_barrier()
        pltpu.sync_copy(acc.at[pl.ds(r0, rpt)],
                        out_ref.at[q, cid, pl.ds(r0, rpt)])
        plsc.subcore_barrier()


def _sc_cnt_body(dst1_ref, out_ref,
                 didx_all, vbuf, zbuf, acc, *, per_tile, chunks, rpt):
    cid = lax.axis_index("c")
    sid = lax.axis_index("s")
    tid = cid * _NS + sid
    r0 = sid * rpt
    pt128 = per_tile // (2 * _EB)
    _zero_slice(zbuf, acc, r0, rpt, 1)
    pltpu.sync_copy(dst1_ref.at[pl.ds(tid * pt128, pt128)], didx_all)

    def vrow(i, c):
        one = jnp.where(lax.iota(jnp.int32, _L) == 0, 1.0, 0.0)
        vbuf[i, pl.ds(0, _L)] = one
        return c

    lax.fori_loop(0, 2 * _EB, vrow, 0)
    plsc.subcore_barrier()

    def chunk(j, carry):
        pltpu.sync_copy(vbuf, acc.at[didx_all.at[j]], add=True)
        return carry

    lax.fori_loop(0, pt128, chunk, 0)
    plsc.subcore_barrier()
    pltpu.sync_copy(acc.at[pl.ds(r0, rpt)], out_ref.at[cid, pl.ds(r0, rpt)])


def kernel(input_feat, edge_index, pseudo, Wg1, mu1, sigma1, Wroot1, b1,
           Wg2, mu2, sigma2, Wroot2, b2):
    n, d = input_feat.shape
    e = pseudo.shape[0]
    k = mu1.shape[0]
    qw = d // _Q
    yw = k * qw

    n_pad = _ceil_to(n, 32 * _EB)
    e_pad = _ceil_to(e, _NC * _NS * _EB * 2)
    e_pad = _ceil_to(e_pad, _BE)
    per_tile = e_pad // (_NC * _NS)
    chunks = per_tile // _EB
    rpt = n_pad // _NS

    f32 = jnp.float32

    x_pad = jnp.pad(input_feat, ((0, n_pad - n), (0, 0)))
    src = jnp.pad(edge_index[0], (0, e_pad - e)).reshape(-1, 2 * _EB)
    dst = jnp.pad(edge_index[1], (0, e_pad - e),
                  constant_values=n).reshape(-1, 2 * _EB)
    pseudo_pad = jnp.pad(pseudo, ((0, e_pad - e), (0, 0)))
    p0rep = jnp.repeat(pseudo_pad[:, 0], _L).reshape(e_pad // 8, 8 * _L)
    p1rep = jnp.repeat(pseudo_pad[:, 1], _L).reshape(e_pad // 8, 8 * _L)
    mup1 = jnp.tile(jnp.pad(mu1.T, ((0, 0), (0, _L - k))), (1, 8))
    sgp1 = jnp.tile(jnp.pad(sigma1.T, ((0, 0), (0, _L - k)),
                            constant_values=1.0), (1, 8))
    mup2 = jnp.tile(jnp.pad(mu2.T, ((0, 0), (0, _L - k))), (1, 8))
    sgp2 = jnp.tile(jnp.pad(sigma2.T, ((0, 0), (0, _L - k)),
                            constant_values=1.0), (1, 8))
    def _perm(wg):
        blk = wg.reshape(d, k, _Q, qw)
        il = jnp.stack([blk[..., :qw // 2], blk[..., qw // 2:]], axis=-1)
        il = il.reshape(d, k, _Q, qw)
        return [il[:, :, q, :].reshape(d, yw) for q in range(_Q)]

    wq1 = _perm(Wg1)
    wq2 = _perm(Wg2)
    b1r = b1.reshape(1, d)
    b2r = b2.reshape(1, d)


    gbr = _BE // 8
    gauss_call = pl.pallas_call(
        functools.partial(_gauss_body, k=k),
        grid=(e_pad // _BE,),
        in_specs=[pl.BlockSpec((gbr, 8 * _L), lambda i: (i, 0)),
                  pl.BlockSpec((gbr, 8 * _L), lambda i: (i, 0)),
                  pl.BlockSpec((2, 8 * _L), lambda i: (0, 0)),
                  pl.BlockSpec((2, 8 * _L), lambda i: (0, 0))],
        out_specs=pl.BlockSpec((gbr, 8 * _L), lambda i: (i, 0)),
        out_shape=jax.ShapeDtypeStruct((e_pad // 8, 8 * _L), f32),
    )
    g16_1 = gauss_call(p0rep, p1rep, mup1, sgp1)
    g16_2 = gauss_call(p0rep, p1rep, mup2, sgp2)

    prep_call = pl.pallas_call(
        _prep_body,
        grid=(n_pad // _BN,),
        in_specs=[pl.BlockSpec((_BN, d), lambda i: (i, 0))] +
                 [pl.BlockSpec((d, yw), lambda i: (0, 0))] * _Q +
                 [pl.BlockSpec((d, d), lambda i: (0, 0)),
                  pl.BlockSpec((1, d), lambda i: (0, 0))],
        out_specs=[pl.BlockSpec((_BN, yw), lambda i: (i, 0))] * _Q +
                  [pl.BlockSpec((_BN, d), lambda i: (i, 0))],
        out_shape=[jax.ShapeDtypeStruct((n_pad, yw), jnp.bfloat16)] * _Q +
                  [jax.ShapeDtypeStruct((n_pad, d), f32)],
    )

    mesh = plsc.VectorSubcoreMesh(core_axis_name="c", subcore_axis_name="s",
                                  num_cores=_NC, num_subcores=_NS)
    sc_params = pltpu.CompilerParams(use_tc_tiling_on_sc=False,
                                     needs_layout_passes=False)
    sc_layer = pl.kernel(
        functools.partial(_sc_layer_body, per_tile=per_tile, chunks=chunks,
                          rpt=rpt, k=k, qw=qw),
        out_type=jax.ShapeDtypeStruct((_Q, _NC, n_pad, qw), f32),
        mesh=mesh,
        scratch_types=[
            pltpu.VMEM((per_tile // (2 * _EB), 2 * _EB), jnp.int32),
            pltpu.VMEM((per_tile // (2 * _EB), 2 * _EB), jnp.int32),
            pltpu.VMEM((_EB // 8, 8 * _L), f32),
            pltpu.VMEM((_EB // 8, 8 * _L), f32),
            pltpu.VMEM((_EB, yw), jnp.bfloat16),
            pltpu.VMEM((_EB, yw), jnp.bfloat16),
            pltpu.VMEM((2 * _EB, qw), f32),
            pltpu.VMEM((_EB, qw), f32),
            pltpu.VMEM_SHARED((n_pad, qw), f32),
            pltpu.SemaphoreType.DMA,
            pltpu.SemaphoreType.DMA,
            pltpu.SemaphoreType.DMA,
            pltpu.SemaphoreType.DMA,
        ],
        compiler_params=sc_params,
    )
    sc_cnt = pl.kernel(
        functools.partial(_sc_cnt_body, per_tile=per_tile, chunks=chunks,
                          rpt=rpt),
        out_type=jax.ShapeDtypeStruct((_NC, n_pad, _L), f32),
        mesh=mesh,
        scratch_types=[
            pltpu.VMEM((per_tile // (2 * _EB), 2 * _EB), jnp.int32),
            pltpu.VMEM((2 * _EB, _L), f32),
            pltpu.VMEM((_EB, _L), f32),
            pltpu.VMEM_SHARED((n_pad, _L), f32),
        ],
        compiler_params=sc_params,
    )

    rpt32 = n_pad // (_NC * _NS)
    fin_scratch = [
        pltpu.VMEM((_Q * _NC, _EB, qw), f32),
        pltpu.VMEM((_Q * _NC, _EB, qw), f32),
        pltpu.VMEM((_NC, _EB, _L), f32),
        pltpu.VMEM((_NC, _EB, _L), f32),
        pltpu.VMEM((_EB, d), f32),
        pltpu.VMEM((_EB, d), f32),
        pltpu.VMEM((_EB, d), f32),
        pltpu.VMEM((_EB, d), f32),
        pltpu.VMEM((_EB, d), f32),
        pltpu.VMEM((_EB, d), f32),
        pltpu.SemaphoreType.DMA,
        pltpu.SemaphoreType.DMA,
    ]
    sc_fin = pl.kernel(
        functools.partial(_sc_fin_body, rpt32=rpt32, qw=qw, with_res=False),
        out_type=jax.ShapeDtypeStruct((n_pad, d), f32),
        mesh=mesh, scratch_types=fin_scratch, compiler_params=sc_params,
    )
    sc_fin_res = pl.kernel(
        functools.partial(_sc_fin_body, rpt32=rpt32, qw=qw, with_res=True),
        out_type=jax.ShapeDtypeStruct((n_pad, d), f32),
        mesh=mesh, scratch_types=fin_scratch, compiler_params=sc_params,
    )

    cnt = sc_cnt(dst)

    y1 = prep_call(x_pad, *wq1, Wroot1, b1r)
    aggs1 = sc_layer(y1[0], y1[1], y1[2], y1[3], src, dst, g16_1)
    h = sc_fin(aggs1, cnt, y1[_Q], y1[_Q])

    y2 = prep_call(h, *wq2, Wroot2, b2r)
    aggs2 = sc_layer(y2[0], y2[1], y2[2], y2[3], src, dst, g16_2)
    out = sc_fin_res(aggs2, cnt, y2[_Q], x_pad)

    return out[:n]
